# baseline (speedup 1.0000x reference)
import functools

import jax
import jax.numpy as jnp
from jax import lax
from jax.experimental import pallas as pl
from jax.experimental.pallas import tpu as pltpu
from jax.experimental.pallas import tpu_sc as plsc

EMB = 64
NC = 2
NS = 16
NW = NC * NS
LANES = 16
IDXW = 128
ROWW = 128
PW = 2048
PSH = 11


def _vsqrt(x):
    i = lax.bitcast_convert_type(x, jnp.int32)
    y = lax.bitcast_convert_type((i >> 1) + jnp.int32(0x1FBD1DF6), jnp.float32)
    y = 0.5 * (y + x / y)
    y = 0.5 * (y + x / y)
    y = 0.5 * (y + x / y)
    return y


def _transpose_merge(tbl_t):
    n = tbl_t.shape[1]
    w = PW
    nblk = pl.cdiv(n, 2 * w)
    last_in = pl.cdiv(n, w) - 1

    def body(lo_ref, hi_ref, out_ref):
        ident = jnp.eye(EMB, dtype=jnp.float32)

        def tpose(x):
            return jax.lax.dot_general(
                x, ident, (((0,), (0,)), ((), ())),
                preferred_element_type=jnp.float32,
                precision=jax.lax.Precision.HIGHEST)

        out_ref[...] = jnp.concatenate(
            [tpose(lo_ref[...]), tpose(hi_ref[...])], axis=1)

    return pl.pallas_call(
        body,
        grid=(nblk,),
        in_specs=[
            pl.BlockSpec((EMB, w), lambda k: (0, 2 * k)),
            pl.BlockSpec((EMB, w), lambda k: (0, jnp.minimum(2 * k + 1, last_in))),
        ],
        out_specs=pl.BlockSpec((w, 2 * EMB), lambda k: (k, 0)),
        out_shape=jax.ShapeDtypeStruct((nblk * w, 2 * EMB), jnp.float32),
    )(tbl_t, tbl_t)


def _make_sc_kernel(tot, chunk, sub):
    ngather = sub // IDXW
    nsub = chunk // sub
    nidx = chunk // IDXW

    mesh = plsc.VectorSubcoreMesh(core_axis_name="c", subcore_axis_name="s")

    @functools.partial(
        pl.kernel,
        out_type=jax.ShapeDtypeStruct((tot,), jnp.float32),
        mesh=mesh,
        scratch_types=dict(
            idx_h=pltpu.VMEM((nidx, IDXW), jnp.int32),
            idx_r=pltpu.VMEM((nidx, IDXW), jnp.int32),
            idx_t=pltpu.VMEM((nidx, IDXW), jnp.int32),
            idx2_h=pltpu.VMEM((nidx, IDXW), jnp.int32),
            idx2_r=pltpu.VMEM((nidx, IDXW), jnp.int32),
            idx2_t=pltpu.VMEM((nidx, IDXW), jnp.int32),
            rows_h=pltpu.VMEM((sub, ROWW), jnp.float32),
            rows_r=pltpu.VMEM((sub, ROWW), jnp.float32),
            rows_t=pltpu.VMEM((sub, ROWW), jnp.float32),
            out_v=pltpu.VMEM((chunk,), jnp.float32),
            sem=pltpu.SemaphoreType.DMA,
        ),
        compiler_params=pltpu.CompilerParams(
            needs_layout_passes=False, use_tc_tiling_on_sc=True),
    )
    def sc_kernel(heads_hbm, rels_hbm, tails_hbm, ent_hbm, rel_hbm, out_hbm,
                  *, idx_h, idx_r, idx_t, idx2_h, idx2_r, idx2_t,
                  rows_h, rows_r, rows_t, out_v, sem):
        wid = lax.axis_index("s") * NC + lax.axis_index("c")
        base = pl.multiple_of(wid * chunk, chunk)
        lane = lax.iota(jnp.int32, 16)

        r0 = pl.multiple_of(base // IDXW, nidx)
        pltpu.sync_copy(heads_hbm.at[pl.ds(r0, nidx)], idx_h)
        pltpu.sync_copy(rels_hbm.at[pl.ds(r0, nidx)], idx_r)
        pltpu.sync_copy(tails_hbm.at[pl.ds(r0, nidx)], idx_t)

        def pair_row(e):
            return ((e >> (PSH + 1)) << PSH) + (e & (PW - 1))

        def shift_body(j, _):
            for q in range(IDXW // LANES):
                sl = pl.ds(q * LANES, LANES)
                idx2_h[j, sl] = pair_row(idx_h[j, sl])
                idx2_r[j, sl] = pair_row(idx_r[j, sl])
                idx2_t[j, sl] = pair_row(idx_t[j, sl])
            return 0

        lax.fori_loop(0, nidx, shift_body, 0)

        for s in range(nsub):
            copies = []
            for j in range(ngather):
                src = s * ngather + j
                dst = pl.ds(j * IDXW, IDXW)
                copies.append(pltpu.async_copy(
                    ent_hbm.at[idx2_h.at[src]], rows_h.at[dst], sem))
                copies.append(pltpu.async_copy(
                    rel_hbm.at[idx2_r.at[src]], rows_r.at[dst], sem))
                copies.append(pltpu.async_copy(
                    ent_hbm.at[idx2_t.at[src]], rows_t.at[dst], sem))
            for cp in copies:
                cp.wait()

            def group_body(g, _, s=s):
                flat = s * sub + g * LANES
                irow = flat // IDXW
                icol = (flat % IDXW) * jnp.int32(1)
                isl = pl.ds(icol, LANES)
                ch = ((idx_h[irow, isl] >> PSH) & 1) * EMB
                cr = ((idx_r[irow, isl] >> PSH) & 1) * EMB
                ct = ((idx_t[irow, isl] >> PSH) & 1) * EMB
                ridx = g * LANES + lane
                acc = jnp.zeros((LANES,), jnp.float32)
                for d in range(EMB):
                    h = plsc.load_gather(rows_h, [ridx, ch + d])
                    r = plsc.load_gather(rows_r, [ridx, cr + d])
                    t = plsc.load_gather(rows_t, [ridx, ct + d])
                    e = h + r - t
                    acc = acc + e * e
                out_v[pl.ds(flat, LANES)] = _vsqrt(acc)
                return 0

            lax.fori_loop(0, sub // LANES, group_body, 0)

        pltpu.sync_copy(out_v, out_hbm.at[pl.ds(base, chunk)])

    return sc_kernel


def kernel(pos_triples, neg_triples, ent_embs, rel_embs):
    b = pos_triples.shape[0]
    tot = 2 * b
    chunk = tot // NW
    sub = min(chunk, 256)

    trip = jnp.concatenate(
        [pos_triples.astype(jnp.int32), neg_triples.astype(jnp.int32)], axis=0)
    heads = trip[:, 0].reshape(tot // IDXW, IDXW)
    rels = trip[:, 1].reshape(tot // IDXW, IDXW)
    tails = trip[:, 2].reshape(tot // IDXW, IDXW)

    ent2 = _transpose_merge(ent_embs.T)
    rel2 = _transpose_merge(rel_embs.T)

    out = _make_sc_kernel(tot, chunk, sub)(heads, rels, tails, ent2, rel2)
    return out[:b], out[b:]


# speedup vs baseline: 1.0878x; 1.0878x over previous
import functools

import jax
import jax.numpy as jnp
from jax import lax
from jax.experimental import pallas as pl
from jax.experimental.pallas import tpu as pltpu
from jax.experimental.pallas import tpu_sc as plsc

EMB = 64
NC = 2
NS = 16
NW = NC * NS
LANES = 16
IDXW = 128
ROWW = 128
PW = 2048
PSH = 11


def _vsqrt(x):
    i = lax.bitcast_convert_type(x, jnp.int32)
    y = lax.bitcast_convert_type((i >> 1) + jnp.int32(0x1FBD1DF6), jnp.float32)
    y = 0.5 * (y + x / y)
    y = 0.5 * (y + x / y)
    y = 0.5 * (y + x / y)
    return y


def _transpose_merge(tbl_t):
    n = tbl_t.shape[1]
    w = PW
    nblk = pl.cdiv(n, 2 * w)
    last_in = pl.cdiv(n, w) - 1

    def body(lo_ref, hi_ref, out_ref):
        out_ref[...] = jnp.concatenate(
            [jnp.transpose(lo_ref[...]), jnp.transpose(hi_ref[...])], axis=1)

    return pl.pallas_call(
        body,
        grid=(nblk,),
        in_specs=[
            pl.BlockSpec((EMB, w), lambda k: (0, 2 * k)),
            pl.BlockSpec((EMB, w), lambda k: (0, jnp.minimum(2 * k + 1, last_in))),
        ],
        out_specs=pl.BlockSpec((w, 2 * EMB), lambda k: (k, 0)),
        out_shape=jax.ShapeDtypeStruct((nblk * w, 2 * EMB), jnp.float32),
    )(tbl_t, tbl_t)


def _depad_merge(tbl):
    n = tbl.shape[0]
    w = PW
    nblk = pl.cdiv(n, 2 * w)
    last_in = pl.cdiv(n, w) - 1

    def body(lo_ref, hi_ref, out_ref):
        out_ref[...] = jnp.concatenate([lo_ref[...], hi_ref[...]], axis=1)

    return pl.pallas_call(
        body,
        grid=(nblk,),
        in_specs=[
            pl.BlockSpec((w, EMB), lambda k: (2 * k, 0)),
            pl.BlockSpec((w, EMB), lambda k: (jnp.minimum(2 * k + 1, last_in), 0)),
        ],
        out_specs=pl.BlockSpec((w, 2 * EMB), lambda k: (k, 0)),
        out_shape=jax.ShapeDtypeStruct((nblk * w, 2 * EMB), jnp.float32),
    )(tbl, tbl)


def _make_sc_kernel(tot, chunk, sub):
    ngather = sub // IDXW
    nsub = chunk // sub
    nidx = chunk // IDXW

    mesh = plsc.VectorSubcoreMesh(core_axis_name="c", subcore_axis_name="s")

    @functools.partial(
        pl.kernel,
        out_type=jax.ShapeDtypeStruct((tot,), jnp.float32),
        mesh=mesh,
        scratch_types=dict(
            idx_h=pltpu.VMEM((nidx, IDXW), jnp.int32),
            idx_r=pltpu.VMEM((nidx, IDXW), jnp.int32),
            idx_t=pltpu.VMEM((nidx, IDXW), jnp.int32),
            idx2_h=pltpu.VMEM((nidx, IDXW), jnp.int32),
            idx2_r=pltpu.VMEM((nidx, IDXW), jnp.int32),
            idx2_t=pltpu.VMEM((nidx, IDXW), jnp.int32),
            rows_h=pltpu.VMEM((sub, ROWW), jnp.float32),
            rows_r=pltpu.VMEM((sub, ROWW), jnp.float32),
            rows_t=pltpu.VMEM((sub, ROWW), jnp.float32),
            out_v=pltpu.VMEM((chunk,), jnp.float32),
            sem=pltpu.SemaphoreType.DMA,
        ),
        compiler_params=pltpu.CompilerParams(
            needs_layout_passes=False, use_tc_tiling_on_sc=True),
    )
    def sc_kernel(heads_hbm, rels_hbm, tails_hbm, ent_hbm, rel_hbm, out_hbm,
                  *, idx_h, idx_r, idx_t, idx2_h, idx2_r, idx2_t,
                  rows_h, rows_r, rows_t, out_v, sem):
        wid = lax.axis_index("s") * NC + lax.axis_index("c")
        base = pl.multiple_of(wid * chunk, chunk)
        lane = lax.iota(jnp.int32, 16)

        r0 = pl.multiple_of(base // IDXW, nidx)
        pltpu.sync_copy(heads_hbm.at[pl.ds(r0, nidx)], idx_h)
        pltpu.sync_copy(rels_hbm.at[pl.ds(r0, nidx)], idx_r)
        pltpu.sync_copy(tails_hbm.at[pl.ds(r0, nidx)], idx_t)

        def pair_row(e):
            return ((e >> (PSH + 1)) << PSH) + (e & (PW - 1))

        def shift_body(j, _):
            for q in range(IDXW // LANES):
                sl = pl.ds(q * LANES, LANES)
                idx2_h[j, sl] = pair_row(idx_h[j, sl])
                idx2_r[j, sl] = pair_row(idx_r[j, sl])
                idx2_t[j, sl] = pair_row(idx_t[j, sl])
            return 0

        lax.fori_loop(0, nidx, shift_body, 0)

        for s in range(nsub):
            copies = []
            for j in range(ngather):
                src = s * ngather + j
                dst = pl.ds(j * IDXW, IDXW)
                copies.append(pltpu.async_copy(
                    ent_hbm.at[idx2_h.at[src]], rows_h.at[dst], sem))
                copies.append(pltpu.async_copy(
                    rel_hbm.at[idx2_r.at[src]], rows_r.at[dst], sem))
                copies.append(pltpu.async_copy(
                    ent_hbm.at[idx2_t.at[src]], rows_t.at[dst], sem))
            for cp in copies:
                cp.wait()

            def group_body(g, _, s=s):
                flat = s * sub + g * LANES
                irow = flat // IDXW
                icol = (flat % IDXW) * jnp.int32(1)
                isl = pl.ds(icol, LANES)
                ch = ((idx_h[irow, isl] >> PSH) & 1) * EMB
                cr = ((idx_r[irow, isl] >> PSH) & 1) * EMB
                ct = ((idx_t[irow, isl] >> PSH) & 1) * EMB
                ridx = g * LANES + lane
                acc = jnp.zeros((LANES,), jnp.float32)
                for d in range(EMB):
                    h = plsc.load_gather(rows_h, [ridx, ch + d])
                    r = plsc.load_gather(rows_r, [ridx, cr + d])
                    t = plsc.load_gather(rows_t, [ridx, ct + d])
                    e = h + r - t
                    acc = acc + e * e
                out_v[pl.ds(flat, LANES)] = _vsqrt(acc)
                return 0

            lax.fori_loop(0, sub // LANES, group_body, 0)

        pltpu.sync_copy(out_v, out_hbm.at[pl.ds(base, chunk)])

    return sc_kernel


def kernel(pos_triples, neg_triples, ent_embs, rel_embs):
    b = pos_triples.shape[0]
    tot = 2 * b
    chunk = tot // NW
    sub = min(chunk, 256)

    trip = jnp.concatenate(
        [pos_triples.astype(jnp.int32), neg_triples.astype(jnp.int32)], axis=0)
    heads = trip[:, 0].reshape(tot // IDXW, IDXW)
    rels = trip[:, 1].reshape(tot // IDXW, IDXW)
    tails = trip[:, 2].reshape(tot // IDXW, IDXW)

    ent2 = _transpose_merge(ent_embs.T)
    rel2 = _depad_merge(rel_embs)

    out = _make_sc_kernel(tot, chunk, sub)(heads, rels, tails, ent2, rel2)
    return out[:b], out[b:]


# speedup vs baseline: 1.5350x; 1.4111x over previous
import functools

import jax
import jax.numpy as jnp
from jax import lax
from jax.experimental import pallas as pl
from jax.experimental.pallas import tpu as pltpu
from jax.experimental.pallas import tpu_sc as plsc

EMB = 64
NC = 2
NS = 16
NW = NC * NS
LANES = 16
IDXW = 128
ROWW = 128
PW = 2048
PSH = 11


def _vsqrt(x):
    i = lax.bitcast_convert_type(x, jnp.int32)
    y = lax.bitcast_convert_type((i >> 1) + jnp.int32(0x1FBD1DF6), jnp.float32)
    y = 0.5 * (y + x / y)
    y = 0.5 * (y + x / y)
    y = 0.5 * (y + x / y)
    return y


def _transpose_merge(tbl_t):
    n = tbl_t.shape[1]
    w = PW
    nblk = pl.cdiv(n, 2 * w)
    last_in = pl.cdiv(n, w) - 1

    def body(lo_ref, hi_ref, out_ref):
        out_ref[...] = jnp.concatenate(
            [jnp.transpose(lo_ref[...]), jnp.transpose(hi_ref[...])], axis=1)

    return pl.pallas_call(
        body,
        grid=(nblk,),
        in_specs=[
            pl.BlockSpec((EMB, w), lambda k: (0, 2 * k)),
            pl.BlockSpec((EMB, w), lambda k: (0, jnp.minimum(2 * k + 1, last_in))),
        ],
        out_specs=pl.BlockSpec((w, 2 * EMB), lambda k: (k, 0)),
        out_shape=jax.ShapeDtypeStruct((nblk * w, 2 * EMB), jnp.float32),
    )(tbl_t, tbl_t)


def _make_sc_kernel(tot, chunk, sub):
    ngather = sub // IDXW
    nsub = chunk // sub
    nidx = chunk // IDXW

    mesh = plsc.VectorSubcoreMesh(core_axis_name="c", subcore_axis_name="s")

    @functools.partial(
        pl.kernel,
        out_type=jax.ShapeDtypeStruct((tot,), jnp.float32),
        mesh=mesh,
        scratch_types=dict(
            idx_h=pltpu.VMEM((nidx, IDXW), jnp.int32),
            idx_r=pltpu.VMEM((nidx, IDXW), jnp.int32),
            idx_t=pltpu.VMEM((nidx, IDXW), jnp.int32),
            idx2_h=pltpu.VMEM((nidx, IDXW), jnp.int32),
            idx2_r=pltpu.VMEM((nidx, IDXW), jnp.int32),
            idx2_t=pltpu.VMEM((nidx, IDXW), jnp.int32),
            rows_h=pltpu.VMEM((sub, ROWW), jnp.float32),
            rows_r=pltpu.VMEM((sub, ROWW), jnp.float32),
            rows_t=pltpu.VMEM((sub, ROWW), jnp.float32),
            out_v=pltpu.VMEM((chunk,), jnp.float32),
            sem=pltpu.SemaphoreType.DMA,
        ),
        compiler_params=pltpu.CompilerParams(
            needs_layout_passes=False, use_tc_tiling_on_sc=True),
    )
    def sc_kernel(heads_hbm, rels_hbm, tails_hbm, ent_hbm, rel_hbm, out_hbm,
                  *, idx_h, idx_r, idx_t, idx2_h, idx2_r, idx2_t,
                  rows_h, rows_r, rows_t, out_v, sem):
        wid = lax.axis_index("s") * NC + lax.axis_index("c")
        base = pl.multiple_of(wid * chunk, chunk)
        lane = lax.iota(jnp.int32, 16)

        r0 = pl.multiple_of(base // IDXW, nidx)
        pltpu.sync_copy(heads_hbm.at[pl.ds(r0, nidx)], idx_h)
        pltpu.sync_copy(rels_hbm.at[pl.ds(r0, nidx)], idx_r)
        pltpu.sync_copy(tails_hbm.at[pl.ds(r0, nidx)], idx_t)

        def pair_row(e):
            return ((e >> (PSH + 1)) << PSH) + (e & (PW - 1))

        def shift_body(j, _):
            for q in range(IDXW // LANES):
                sl = pl.ds(q * LANES, LANES)
                idx2_h[j, sl] = pair_row(idx_h[j, sl])
                idx2_r[j, sl] = pair_row(idx_r[j, sl])
                idx2_t[j, sl] = pair_row(idx_t[j, sl])
            return 0

        lax.fori_loop(0, nidx, shift_body, 0)

        for s in range(nsub):
            copies = []
            for j in range(ngather):
                src = s * ngather + j
                dst = pl.ds(j * IDXW, IDXW)
                copies.append(pltpu.async_copy(
                    ent_hbm.at[idx2_h.at[src]], rows_h.at[dst], sem))
                copies.append(pltpu.async_copy(
                    rel_hbm.at[idx2_r.at[src]], rows_r.at[dst], sem))
                copies.append(pltpu.async_copy(
                    ent_hbm.at[idx2_t.at[src]], rows_t.at[dst], sem))
            for cp in copies:
                cp.wait()

            def group_body(g, _, s=s):
                flat = s * sub + g * LANES
                irow = flat // IDXW
                icol = (flat % IDXW) * jnp.int32(1)
                isl = pl.ds(icol, LANES)
                ch = ((idx_h[irow, isl] >> PSH) & 1) * EMB
                cr = ((idx_r[irow, isl] >> PSH) & 1) * EMB
                ct = ((idx_t[irow, isl] >> PSH) & 1) * EMB
                ridx = g * LANES + lane
                acc = jnp.zeros((LANES,), jnp.float32)
                for d in range(EMB):
                    h = plsc.load_gather(rows_h, [ridx, ch + d])
                    r = plsc.load_gather(rows_r, [ridx, cr + d])
                    t = plsc.load_gather(rows_t, [ridx, ct + d])
                    e = h + r - t
                    acc = acc + e * e
                out_v[pl.ds(flat, LANES)] = _vsqrt(acc)
                return 0

            lax.fori_loop(0, sub // LANES, group_body, 0)

        pltpu.sync_copy(out_v, out_hbm.at[pl.ds(base, chunk)])

    return sc_kernel


def kernel(pos_triples, neg_triples, ent_embs, rel_embs):
    b = pos_triples.shape[0]
    tot = 2 * b
    chunk = tot // NW
    sub = min(chunk, 256)

    trip = jnp.concatenate(
        [pos_triples.astype(jnp.int32), neg_triples.astype(jnp.int32)], axis=0)
    heads = trip[:, 0].reshape(tot // IDXW, IDXW)
    rels = trip[:, 1].reshape(tot // IDXW, IDXW)
    tails = trip[:, 2].reshape(tot // IDXW, IDXW)

    ent2 = _transpose_merge(ent_embs.T)
    rel2 = _transpose_merge(rel_embs.T)

    out = _make_sc_kernel(tot, chunk, sub)(heads, rels, tails, ent2, rel2)
    return out[:b], out[b:]


# speedup vs baseline: 1.8610x; 1.2124x over previous
import functools

import jax
import jax.numpy as jnp
from jax import lax
from jax.experimental import pallas as pl
from jax.experimental.pallas import tpu as pltpu
from jax.experimental.pallas import tpu_sc as plsc

EMB = 64
NC = 2
NS = 16
NW = NC * NS
LANES = 16
IDXW = 128
ROWW = 128
PW = 4096
PSH = 12


def _vsqrt(x):
    i = lax.bitcast_convert_type(x, jnp.int32)
    y = lax.bitcast_convert_type((i >> 1) + jnp.int32(0x1FBD1DF6), jnp.float32)
    y = 0.5 * (y + x / y)
    y = 0.5 * (y + x / y)
    y = 0.5 * (y + x / y)
    return y


def _transpose_merge(tbl_t):
    n = tbl_t.shape[1]
    w = PW
    nblk = pl.cdiv(n, 2 * w)
    last_in = pl.cdiv(n, w) - 1

    def body(lo_ref, hi_ref, out_ref):
        out_ref[...] = jnp.concatenate(
            [jnp.transpose(lo_ref[...]), jnp.transpose(hi_ref[...])], axis=1)

    return pl.pallas_call(
        body,
        grid=(nblk,),
        in_specs=[
            pl.BlockSpec((EMB, w), lambda k: (0, 2 * k)),
            pl.BlockSpec((EMB, w), lambda k: (0, jnp.minimum(2 * k + 1, last_in))),
        ],
        out_specs=pl.BlockSpec((w, 2 * EMB), lambda k: (k, 0)),
        out_shape=jax.ShapeDtypeStruct((nblk * w, 2 * EMB), jnp.float32),
    )(tbl_t, tbl_t)


def _make_sc_kernel(tot, chunk, sub):
    ngather = sub // IDXW
    nsub = chunk // sub
    nidx = chunk // IDXW

    mesh = plsc.VectorSubcoreMesh(core_axis_name="c", subcore_axis_name="s")

    @functools.partial(
        pl.kernel,
        out_type=jax.ShapeDtypeStruct((tot,), jnp.float32),
        mesh=mesh,
        scratch_types=dict(
            idx_h=pltpu.VMEM((nidx, IDXW), jnp.int32),
            idx_r=pltpu.VMEM((nidx, IDXW), jnp.int32),
            idx_t=pltpu.VMEM((nidx, IDXW), jnp.int32),
            idx2_h=pltpu.VMEM((nidx, IDXW), jnp.int32),
            idx2_r=pltpu.VMEM((nidx, IDXW), jnp.int32),
            idx2_t=pltpu.VMEM((nidx, IDXW), jnp.int32),
            rows_h=pltpu.VMEM((sub, ROWW), jnp.float32),
            rows_r=pltpu.VMEM((sub, ROWW), jnp.float32),
            rows_t=pltpu.VMEM((sub, ROWW), jnp.float32),
            out_v=pltpu.VMEM((chunk,), jnp.float32),
            sem=pltpu.SemaphoreType.DMA,
        ),
        compiler_params=pltpu.CompilerParams(
            needs_layout_passes=False, use_tc_tiling_on_sc=True),
    )
    def sc_kernel(heads_hbm, rels_hbm, tails_hbm, ent_hbm, rel_hbm, out_hbm,
                  *, idx_h, idx_r, idx_t, idx2_h, idx2_r, idx2_t,
                  rows_h, rows_r, rows_t, out_v, sem):
        wid = lax.axis_index("s") * NC + lax.axis_index("c")
        base = pl.multiple_of(wid * chunk, chunk)
        lane = lax.iota(jnp.int32, 16)

        r0 = pl.multiple_of(base // IDXW, nidx)
        pltpu.sync_copy(heads_hbm.at[pl.ds(r0, nidx)], idx_h)
        pltpu.sync_copy(rels_hbm.at[pl.ds(r0, nidx)], idx_r)
        pltpu.sync_copy(tails_hbm.at[pl.ds(r0, nidx)], idx_t)

        def pair_row(e):
            return ((e >> (PSH + 1)) << PSH) + (e & (PW - 1))

        def shift_body(j, _):
            for q in range(IDXW // LANES):
                sl = pl.ds(q * LANES, LANES)
                idx2_h[j, sl] = pair_row(idx_h[j, sl])
                idx2_r[j, sl] = pair_row(idx_r[j, sl])
                idx2_t[j, sl] = pair_row(idx_t[j, sl])
            return 0

        lax.fori_loop(0, nidx, shift_body, 0)

        for s in range(nsub):
            copies = []
            for j in range(ngather):
                src = s * ngather + j
                dst = pl.ds(j * IDXW, IDXW)
                copies.append(pltpu.async_copy(
                    ent_hbm.at[idx2_h.at[src]], rows_h.at[dst], sem))
                copies.append(pltpu.async_copy(
                    rel_hbm.at[idx2_r.at[src]], rows_r.at[dst], sem))
                copies.append(pltpu.async_copy(
                    ent_hbm.at[idx2_t.at[src]], rows_t.at[dst], sem))
            for cp in copies:
                cp.wait()

            def group_body(g, _, s=s):
                flat = s * sub + g * LANES
                irow = flat // IDXW
                icol = (flat % IDXW) * jnp.int32(1)
                isl = pl.ds(icol, LANES)
                ch = ((idx_h[irow, isl] >> PSH) & 1) * EMB
                cr = ((idx_r[irow, isl] >> PSH) & 1) * EMB
                ct = ((idx_t[irow, isl] >> PSH) & 1) * EMB
                ridx = g * LANES + lane
                acc = jnp.zeros((LANES,), jnp.float32)
                for d in range(EMB):
                    h = plsc.load_gather(rows_h, [ridx, ch + d])
                    r = plsc.load_gather(rows_r, [ridx, cr + d])
                    t = plsc.load_gather(rows_t, [ridx, ct + d])
                    e = h + r - t
                    acc = acc + e * e
                out_v[pl.ds(flat, LANES)] = _vsqrt(acc)
                return 0

            lax.fori_loop(0, sub // LANES, group_body, 0)

        pltpu.sync_copy(out_v, out_hbm.at[pl.ds(base, chunk)])

    return sc_kernel


def kernel(pos_triples, neg_triples, ent_embs, rel_embs):
    b = pos_triples.shape[0]
    tot = 2 * b
    chunk = tot // NW
    sub = min(chunk, 256)

    trip = jnp.concatenate(
        [pos_triples.astype(jnp.int32), neg_triples.astype(jnp.int32)], axis=0)
    heads = trip[:, 0].reshape(tot // IDXW, IDXW)
    rels = trip[:, 1].reshape(tot // IDXW, IDXW)
    tails = trip[:, 2].reshape(tot // IDXW, IDXW)

    ent2 = _transpose_merge(ent_embs.T)
    rel2 = _transpose_merge(rel_embs.T)

    out = _make_sc_kernel(tot, chunk, sub)(heads, rels, tails, ent2, rel2)
    return out[:b], out[b:]


# speedup vs baseline: 2.0739x; 1.1144x over previous
import functools

import jax
import jax.numpy as jnp
from jax import lax
from jax.experimental import pallas as pl
from jax.experimental.pallas import tpu as pltpu
from jax.experimental.pallas import tpu_sc as plsc

EMB = 64
NC = 2
NS = 16
NW = NC * NS
LANES = 16
IDXW = 128
ROWW = 128
PW = 8192
PSH = 13


def _vsqrt(x):
    i = lax.bitcast_convert_type(x, jnp.int32)
    y = lax.bitcast_convert_type((i >> 1) + jnp.int32(0x1FBD1DF6), jnp.float32)
    y = 0.5 * (y + x / y)
    y = 0.5 * (y + x / y)
    y = 0.5 * (y + x / y)
    return y


def _transpose_merge(tbl_t):
    n = tbl_t.shape[1]
    w = PW
    nblk = pl.cdiv(n, 2 * w)
    last_in = pl.cdiv(n, w) - 1

    def body(lo_ref, hi_ref, out_ref):
        out_ref[...] = jnp.concatenate(
            [jnp.transpose(lo_ref[...]), jnp.transpose(hi_ref[...])], axis=1)

    return pl.pallas_call(
        body,
        grid=(nblk,),
        in_specs=[
            pl.BlockSpec((EMB, w), lambda k: (0, 2 * k)),
            pl.BlockSpec((EMB, w), lambda k: (0, jnp.minimum(2 * k + 1, last_in))),
        ],
        out_specs=pl.BlockSpec((w, 2 * EMB), lambda k: (k, 0)),
        out_shape=jax.ShapeDtypeStruct((nblk * w, 2 * EMB), jnp.float32),
    )(tbl_t, tbl_t)


def _make_sc_kernel(tot, chunk, sub):
    ngather = sub // IDXW
    nsub = chunk // sub
    nidx = chunk // IDXW

    mesh = plsc.VectorSubcoreMesh(core_axis_name="c", subcore_axis_name="s")

    @functools.partial(
        pl.kernel,
        out_type=jax.ShapeDtypeStruct((tot,), jnp.float32),
        mesh=mesh,
        scratch_types=dict(
            idx_h=pltpu.VMEM((nidx, IDXW), jnp.int32),
            idx_r=pltpu.VMEM((nidx, IDXW), jnp.int32),
            idx_t=pltpu.VMEM((nidx, IDXW), jnp.int32),
            idx2_h=pltpu.VMEM((nidx, IDXW), jnp.int32),
            idx2_r=pltpu.VMEM((nidx, IDXW), jnp.int32),
            idx2_t=pltpu.VMEM((nidx, IDXW), jnp.int32),
            rows_h=pltpu.VMEM((sub, ROWW), jnp.float32),
            rows_r=pltpu.VMEM((sub, ROWW), jnp.float32),
            rows_t=pltpu.VMEM((sub, ROWW), jnp.float32),
            out_v=pltpu.VMEM((chunk,), jnp.float32),
            sem=pltpu.SemaphoreType.DMA,
        ),
        compiler_params=pltpu.CompilerParams(
            needs_layout_passes=False, use_tc_tiling_on_sc=True),
    )
    def sc_kernel(heads_hbm, rels_hbm, tails_hbm, ent_hbm, rel_hbm, out_hbm,
                  *, idx_h, idx_r, idx_t, idx2_h, idx2_r, idx2_t,
                  rows_h, rows_r, rows_t, out_v, sem):
        wid = lax.axis_index("s") * NC + lax.axis_index("c")
        base = pl.multiple_of(wid * chunk, chunk)
        lane = lax.iota(jnp.int32, 16)

        r0 = pl.multiple_of(base // IDXW, nidx)
        pltpu.sync_copy(heads_hbm.at[pl.ds(r0, nidx)], idx_h)
        pltpu.sync_copy(rels_hbm.at[pl.ds(r0, nidx)], idx_r)
        pltpu.sync_copy(tails_hbm.at[pl.ds(r0, nidx)], idx_t)

        def pair_row(e):
            return ((e >> (PSH + 1)) << PSH) + (e & (PW - 1))

        def shift_body(j, _):
            for q in range(IDXW // LANES):
                sl = pl.ds(q * LANES, LANES)
                idx2_h[j, sl] = pair_row(idx_h[j, sl])
                idx2_r[j, sl] = pair_row(idx_r[j, sl])
                idx2_t[j, sl] = pair_row(idx_t[j, sl])
            return 0

        lax.fori_loop(0, nidx, shift_body, 0)

        for s in range(nsub):
            copies = []
            for j in range(ngather):
                src = s * ngather + j
                dst = pl.ds(j * IDXW, IDXW)
                copies.append(pltpu.async_copy(
                    ent_hbm.at[idx2_h.at[src]], rows_h.at[dst], sem))
                copies.append(pltpu.async_copy(
                    rel_hbm.at[idx2_r.at[src]], rows_r.at[dst], sem))
                copies.append(pltpu.async_copy(
                    ent_hbm.at[idx2_t.at[src]], rows_t.at[dst], sem))
            for cp in copies:
                cp.wait()

            def group_body(g, _, s=s):
                flat = s * sub + g * LANES
                irow = flat // IDXW
                icol = (flat % IDXW) * jnp.int32(1)
                isl = pl.ds(icol, LANES)
                ch = ((idx_h[irow, isl] >> PSH) & 1) * EMB
                cr = ((idx_r[irow, isl] >> PSH) & 1) * EMB
                ct = ((idx_t[irow, isl] >> PSH) & 1) * EMB
                ridx = g * LANES + lane
                acc = jnp.zeros((LANES,), jnp.float32)
                for d in range(EMB):
                    h = plsc.load_gather(rows_h, [ridx, ch + d])
                    r = plsc.load_gather(rows_r, [ridx, cr + d])
                    t = plsc.load_gather(rows_t, [ridx, ct + d])
                    e = h + r - t
                    acc = acc + e * e
                out_v[pl.ds(flat, LANES)] = _vsqrt(acc)
                return 0

            lax.fori_loop(0, sub // LANES, group_body, 0)

        pltpu.sync_copy(out_v, out_hbm.at[pl.ds(base, chunk)])

    return sc_kernel


def kernel(pos_triples, neg_triples, ent_embs, rel_embs):
    b = pos_triples.shape[0]
    tot = 2 * b
    chunk = tot // NW
    sub = min(chunk, 256)

    trip = jnp.concatenate(
        [pos_triples.astype(jnp.int32), neg_triples.astype(jnp.int32)], axis=0)
    heads = trip[:, 0].reshape(tot // IDXW, IDXW)
    rels = trip[:, 1].reshape(tot // IDXW, IDXW)
    tails = trip[:, 2].reshape(tot // IDXW, IDXW)

    ent2 = _transpose_merge(ent_embs.T)
    rel2 = _transpose_merge(rel_embs.T)

    out = _make_sc_kernel(tot, chunk, sub)(heads, rels, tails, ent2, rel2)
    return out[:b], out[b:]
